# Initial kernel scaffold; baseline (speedup 1.0000x reference)
import jax
import jax.numpy as jnp
from jax.experimental import pallas as pl


def kernel(user_index, item_index, adj_ent, adj_rel, ent, rel, desc_tab, W_agg, b_agg, W_dr, b_dr, W1, b1, W2, b2, W3, b3, W_nm, b_nm):
    raise NotImplementedError("write your pallas kernel here")


# speedup vs baseline: 4.8742x; 4.8742x over previous
import functools

import jax
import jax.numpy as jnp
from jax import lax
from jax.experimental import pallas as pl
from jax.experimental.pallas import tpu as pltpu
from jax.experimental.pallas import tpu_sc as plsc

B = 4096
DIM = 128
DESC_DIM = 768
NNB = 8
NW = 32
PB = B // NW
BLK = 128
NB = B // BLK

N_IDX = PB * (1 + NNB + NNB * NNB) + 16
N_REL = PB * (NNB + NNB * NNB) + 16


def _sc_body(ui_hbm, ii_hbm, adj_pad, ent, desc_tab,
             rel1_u, rel2_u, rel1_i, rel2_i,
             e0_u, e1_u, e2_u, e0_i, e1_i, e2_i, d_u, d_i,
             idxall, relbuf, abuf, ebuf, dbuf, sem):
    nc = 2
    wid = lax.axis_index("s") * nc + lax.axis_index("c")
    base = pl.multiple_of(wid * PB, PB)

    def extract(dst_ref, dst0, col):
        def ex(j, _):
            v = abuf[j, col:col + 16]
            dst_ref[pl.ds(pl.multiple_of(dst0 + 8 * j, 8), 16)] = v
            return 0
        lax.fori_loop(0, PB, ex, 0)

    def do_side(base_hbm, rel1_o, rel2_o, e0_o, e1_o, e2_o, d_o):
        pltpu.sync_copy(base_hbm.at[pl.ds(base, PB)], idxall.at[pl.ds(0, PB)])
        pltpu.async_copy(adj_pad.at[idxall.at[pl.ds(0, PB)]], abuf, sem).wait()
        extract(idxall, PB, 0)
        extract(relbuf, 0, 8)

        def hop2(c, _):
            off = pl.multiple_of(PB + PB * c, PB)
            pltpu.async_copy(adj_pad.at[idxall.at[pl.ds(off, PB)]], abuf,
                             sem).wait()
            extract(idxall, PB * (1 + NNB) + 1024 * c, 0)
            extract(relbuf, 1024 + 1024 * c, 8)
            return 0
        lax.fori_loop(0, NNB, hop2, 0)
        pltpu.sync_copy(
            relbuf.at[pl.ds(0, PB * NNB)],
            rel1_o.at[pl.ds(pl.multiple_of(wid * PB * NNB, PB), PB * NNB)])
        pltpu.sync_copy(
            relbuf.at[pl.ds(PB * NNB, PB * NNB * NNB)],
            rel2_o.at[pl.ds(pl.multiple_of(wid * PB * NNB * NNB, PB),
                            PB * NNB * NNB)])

        pltpu.async_copy(ent.at[idxall.at[pl.ds(0, PB)]], ebuf, sem).wait()
        pltpu.sync_copy(ebuf, e0_o.at[pl.ds(base, PB)])

        def g1(c, _):
            off = pl.multiple_of(PB + PB * c, PB)
            pltpu.async_copy(ent.at[idxall.at[pl.ds(off, PB)]], ebuf,
                             sem).wait()
            dst = pl.multiple_of(wid * PB * NNB + PB * c, PB)
            pltpu.sync_copy(ebuf, e1_o.at[pl.ds(dst, PB)])
            return 0
        lax.fori_loop(0, NNB, g1, 0)

        def g2(c, _):
            off = pl.multiple_of(PB * (1 + NNB) + PB * c, PB)
            pltpu.async_copy(ent.at[idxall.at[pl.ds(off, PB)]], ebuf,
                             sem).wait()
            dst = pl.multiple_of(wid * PB * NNB * NNB + PB * c, PB)
            pltpu.sync_copy(ebuf, e2_o.at[pl.ds(dst, PB)])
            return 0
        lax.fori_loop(0, NNB * NNB, g2, 0)

        def gd(c, _):
            off = pl.multiple_of(16 * c, 16)
            pltpu.async_copy(desc_tab.at[idxall.at[pl.ds(off, 16)]], dbuf,
                             sem).wait()
            dst = pl.multiple_of(wid * PB + 16 * c, 16)
            pltpu.sync_copy(dbuf, d_o.at[pl.ds(dst, 16)])
            return 0
        lax.fori_loop(0, PB // 16, gd, 0)

    do_side(ui_hbm, rel1_u, rel2_u, e0_u, e1_u, e2_u, d_u)
    do_side(ii_hbm, rel1_i, rel2_i, e0_i, e1_i, e2_i, d_i)


def _sc_gather(user_index, item_index, adj_pad, ent, desc_tab):
    i32, f32 = jnp.int32, jnp.float32
    out_type = [
        jax.ShapeDtypeStruct((B * NNB,), i32),
        jax.ShapeDtypeStruct((B * NNB * NNB,), i32),
        jax.ShapeDtypeStruct((B * NNB,), i32),
        jax.ShapeDtypeStruct((B * NNB * NNB,), i32),
        jax.ShapeDtypeStruct((B, DIM), f32),
        jax.ShapeDtypeStruct((B * NNB, DIM), f32),
        jax.ShapeDtypeStruct((B * NNB * NNB, DIM), f32),
        jax.ShapeDtypeStruct((B, DIM), f32),
        jax.ShapeDtypeStruct((B * NNB, DIM), f32),
        jax.ShapeDtypeStruct((B * NNB * NNB, DIM), f32),
        jax.ShapeDtypeStruct((B, DESC_DIM), f32),
        jax.ShapeDtypeStruct((B, DESC_DIM), f32),
    ]
    scratch = [
        pltpu.VMEM((N_IDX,), i32),
        pltpu.VMEM((N_REL,), i32),
        pltpu.VMEM((PB, 128), i32),
        pltpu.VMEM((PB, DIM), f32),
        pltpu.VMEM((16, DESC_DIM), f32),
        pltpu.SemaphoreType.DMA,
    ]
    mesh = plsc.VectorSubcoreMesh(core_axis_name="c", subcore_axis_name="s")
    fn = pl.kernel(_sc_body, out_type=out_type, mesh=mesh,
                   scratch_types=scratch)
    return fn(user_index, item_index, adj_pad, ent, desc_tab)


def _tc_body(e0u, e1u, e2u, e0i, e1i, e2i, r1u, r2u, r1i, r2i, du, di,
             rel, wagg, bagg, wdr, bdr, w1, b1, w2, b2, w3, b3, wnm, bnm,
             og, od):
    wt = wagg[0:DIM, :]
    wb = wagg[DIM:2 * DIM, :]
    ba = bagg[...]
    rel_m = rel[...]

    def gather_scores(s_rel, idx):
        out = jnp.zeros(idx.shape, jnp.float32)
        for k in range(32):
            out = out + jnp.where(idx == k, s_rel[:, k:k + 1], 0.0)
        return out

    def softmax_rows(s):
        m = jnp.max(s, axis=-1, keepdims=True)
        e = jnp.exp(s - m)
        return e / jnp.sum(e, axis=-1, keepdims=True)

    def aggregate(side, e0, e1, e2, r1, r2):
        s_rel = lax.dot_general(side, rel_m, (((1,), (1,)), ((), ())),
                                preferred_element_type=jnp.float32)
        s2 = gather_scores(s_rel, r2)
        e1m = e1.reshape(BLK, NNB, DIM)
        e2m = e2.reshape(BLK, NNB * NNB, DIM)
        aggs = []
        for i in range(NNB):
            w2g = softmax_rows(s2[:, NNB * i:NNB * (i + 1)])
            a = w2g[:, 0:1] * e2m[:, NNB * i, :]
            for n in range(1, NNB):
                a = a + w2g[:, n:n + 1] * e2m[:, NNB * i + n, :]
            aggs.append(a)
        x1 = jnp.concatenate([e1m[:, i, :] for i in range(NNB)], axis=0)
        a1 = jnp.concatenate(aggs, axis=0)
        h1 = jax.nn.sigmoid(
            jnp.dot(x1, wt, preferred_element_type=jnp.float32)
            + jnp.dot(a1, wb, preferred_element_type=jnp.float32) + ba)
        w1a = softmax_rows(gather_scores(s_rel, r1))
        agg1 = w1a[:, 0:1] * e1m[:, 0, :]
        aggf = w1a[:, 0:1] * h1[0:BLK, :]
        for i in range(1, NNB):
            agg1 = agg1 + w1a[:, i:i + 1] * e1m[:, i, :]
            aggf = aggf + w1a[:, i:i + 1] * h1[BLK * i:BLK * (i + 1), :]
        h0 = jax.nn.sigmoid(
            jnp.dot(e0, wt, preferred_element_type=jnp.float32)
            + jnp.dot(agg1, wb, preferred_element_type=jnp.float32) + ba)
        return jnp.tanh(
            jnp.dot(h0, wt, preferred_element_type=jnp.float32)
            + jnp.dot(aggf, wb, preferred_element_type=jnp.float32) + ba)

    side_u = e0u[...]
    item_graph = aggregate(side_u, e0i[...], e1i[...], e2i[...],
                           r1i[...], r2i[...])
    user_graph = aggregate(item_graph, side_u, e1u[...], e2u[...],
                           r1u[...], r2u[...])
    og[...] = jax.nn.sigmoid(jnp.sum(user_graph * item_graph, axis=1))

    ud = jax.nn.relu(jnp.dot(du[...], wdr[...],
                             preferred_element_type=jnp.float32) + bdr[...])
    idd = jax.nn.relu(jnp.dot(di[...], wdr[...],
                              preferred_element_type=jnp.float32) + bdr[...])
    nl = jax.nn.relu(
        jnp.dot(ud, w1[0:DIM, :], preferred_element_type=jnp.float32)
        + jnp.dot(idd, w1[DIM:2 * DIM, :], preferred_element_type=jnp.float32)
        + b1[...])
    nl = jax.nn.relu(jnp.dot(nl, w2[...],
                             preferred_element_type=jnp.float32) + b2[...])
    nl = jax.nn.relu(jnp.dot(nl, w3[...],
                             preferred_element_type=jnp.float32) + b3[...])
    lmul = ud * idd
    sd = (jnp.sum(lmul * wnm[:, 0:DIM], axis=1)
          + jnp.sum(nl * wnm[:, DIM:DIM + DIM // 2], axis=1) + bnm[0, 0])
    od[...] = jax.nn.sigmoid(sd)


def _tc_compute(e0u, e1u, e2u, e0i, e1i, e2i, r1u, r2u, r1i, r2i, du, di,
                rel, wagg, bagg, wdr, bdr, w1, b1, w2, b2, w3, b3, wnm, bnm):
    f32 = jnp.float32

    def blk(shape, imap):
        return pl.BlockSpec(shape, imap)

    row = lambda i: (i, 0)
    whole = lambda i: (0, 0)
    in_specs = [
        blk((BLK, DIM), row), blk((BLK * NNB, DIM), row),
        blk((BLK * NNB * NNB, DIM), row),
        blk((BLK, DIM), row), blk((BLK * NNB, DIM), row),
        blk((BLK * NNB * NNB, DIM), row),
        blk((BLK, NNB), row), blk((BLK, NNB * NNB), row),
        blk((BLK, NNB), row), blk((BLK, NNB * NNB), row),
        blk((BLK, DESC_DIM), row), blk((BLK, DESC_DIM), row),
        blk((32, DIM), whole), blk((2 * DIM, DIM), whole),
        blk((1, DIM), whole), blk((DESC_DIM, DIM), whole),
        blk((1, DIM), whole), blk((2 * DIM, 2 * DIM), whole),
        blk((1, 2 * DIM), whole), blk((2 * DIM, DIM), whole),
        blk((1, DIM), whole), blk((DIM, DIM // 2), whole),
        blk((1, DIM // 2), whole), blk((1, DIM + DIM // 2), whole),
        blk((1, 1), whole),
    ]
    out_specs = [pl.BlockSpec((BLK,), lambda i: (i,)),
                 pl.BlockSpec((BLK,), lambda i: (i,))]
    out_shape = [jax.ShapeDtypeStruct((B,), f32),
                 jax.ShapeDtypeStruct((B,), f32)]
    return pl.pallas_call(
        _tc_body, grid=(NB,), in_specs=in_specs, out_specs=out_specs,
        out_shape=out_shape,
    )(e0u, e1u, e2u, e0i, e1i, e2i, r1u, r2u, r1i, r2i, du, di,
      rel, wagg, bagg, wdr, bdr, w1, b1, w2, b2, w3, b3, wnm, bnm)


def kernel(user_index, item_index, adj_ent, adj_rel, ent, rel, desc_tab,
           W_agg, b_agg, W_dr, b_dr, W1, b1, W2, b2, W3, b3, W_nm, b_nm):
    num_ent = adj_ent.shape[0]
    adj_pad = jnp.concatenate(
        [adj_ent, adj_rel,
         jnp.zeros((num_ent, 128 - 2 * NNB), jnp.int32)], axis=1)
    (rel1_u, rel2_u, rel1_i, rel2_i,
     e0u, e1u, e2u, e0i, e1i, e2i, du, di) = _sc_gather(
        user_index, item_index, adj_pad, ent, desc_tab)
    r1u = rel1_u.reshape(B, NNB)
    r2u = rel2_u.reshape(B, NNB * NNB)
    r1i = rel1_i.reshape(B, NNB)
    r2i = rel2_i.reshape(B, NNB * NNB)
    og, od = _tc_compute(
        e0u, e1u, e2u, e0i, e1i, e2i, r1u, r2u, r1i, r2i, du, di,
        rel, W_agg, b_agg.reshape(1, DIM), W_dr, b_dr.reshape(1, DIM),
        W1, b1.reshape(1, 2 * DIM), W2, b2.reshape(1, DIM),
        W3, b3.reshape(1, DIM // 2), W_nm.reshape(1, DIM + DIM // 2),
        b_nm.reshape(1, 1))
    return og, od


# speedup vs baseline: 5.3253x; 1.0925x over previous
import functools

import jax
import jax.numpy as jnp
from jax import lax
from jax.experimental import pallas as pl
from jax.experimental.pallas import tpu as pltpu
from jax.experimental.pallas import tpu_sc as plsc

B = 4096
DIM = 128
DESC_DIM = 768
NNB = 8
NW = 32
PB = B // NW
BLK = 128
NB = B // BLK

N_IDX = PB * (1 + NNB + NNB * NNB) + 16
N_IDXT = PB * (NNB + NNB * NNB)
N_REL = PB * (NNB + NNB * NNB) + 16


def _sc_body(ui_hbm, ii_hbm, adj_pad, ent, desc_tab,
             rel1_u, rel2_u, rel1_i, rel2_i,
             e0_u, e1_u, e2_u, e0_i, e1_i, e2_i, d_u, d_i,
             idxall, relbuf, abuf, ebuf, dbuf, sem, wsem):
    nc = 2
    wid = lax.axis_index("s") * nc + lax.axis_index("c")
    base = pl.multiple_of(wid * PB, PB)

    def extract(dst_ref, dst0, col):
        def ex(j, _):
            v = abuf[j, col:col + 16]
            dst_ref[pl.ds(pl.multiple_of(dst0 + 8 * j, 8), 16)] = v
            return 0
        lax.fori_loop(0, PB, ex, 0)

    def do_side(base_hbm, rel1_o, rel2_o, e0_o, e1_o, e2_o, d_o):
        pltpu.sync_copy(base_hbm.at[pl.ds(base, PB)], idxall.at[pl.ds(0, PB)])
        pltpu.async_copy(adj_pad.at[idxall.at[pl.ds(0, PB)]], abuf, sem).wait()
        extract(idxall, PB, 0)
        extract(relbuf, 0, 8)

        def hop2(c, _):
            off = pl.multiple_of(PB + PB * c, PB)
            pltpu.async_copy(adj_pad.at[idxall.at[pl.ds(off, PB)]], abuf,
                             sem).wait()
            extract(idxall, PB * (1 + NNB) + 1024 * c, 0)
            extract(relbuf, 1024 + 1024 * c, 8)
            return 0
        lax.fori_loop(0, NNB, hop2, 0)
        pltpu.sync_copy(
            relbuf.at[pl.ds(0, PB * NNB)],
            rel1_o.at[pl.ds(pl.multiple_of(wid * PB * NNB, PB), PB * NNB)])
        pltpu.sync_copy(
            relbuf.at[pl.ds(PB * NNB, PB * NNB * NNB)],
            rel2_o.at[pl.ds(pl.multiple_of(wid * PB * NNB * NNB, PB),
                            PB * NNB * NNB)])

        pltpu.async_copy(ent.at[idxall.at[pl.ds(0, PB)]], ebuf, sem).wait()
        pltpu.sync_copy(ebuf, e0_o.at[pl.ds(base, PB)])

        def g1(c, _):
            off = pl.multiple_of(PB + PB * c, PB)
            pltpu.async_copy(ent.at[idxall.at[pl.ds(off, PB)]], ebuf,
                             sem).wait()
            hs = [pltpu.async_copy(ebuf.at[pl.ds(NNB * t, NNB), :],
                                   e1_o.at[:, base + 16 * c + t, :], wsem)
                  for t in range(16)]
            for h in hs:
                h.wait()
            return 0
        lax.fori_loop(0, NNB, g1, 0)

        def g2(c, _):
            off = pl.multiple_of(PB * (1 + NNB) + PB * c, PB)
            pltpu.async_copy(ent.at[idxall.at[pl.ds(off, PB)]], ebuf,
                             sem).wait()
            h0 = pltpu.async_copy(ebuf.at[pl.ds(0, 64), :],
                                  e2_o.at[:, base + 2 * c, :], wsem)
            h1 = pltpu.async_copy(ebuf.at[pl.ds(64, 64), :],
                                  e2_o.at[:, base + 2 * c + 1, :], wsem)
            h0.wait()
            h1.wait()
            return 0
        lax.fori_loop(0, NNB * NNB, g2, 0)

        def gd(c, _):
            off = pl.multiple_of(16 * c, 16)
            pltpu.async_copy(desc_tab.at[idxall.at[pl.ds(off, 16)]], dbuf,
                             sem).wait()
            dst = pl.multiple_of(wid * PB + 16 * c, 16)
            pltpu.sync_copy(dbuf, d_o.at[pl.ds(dst, 16)])
            return 0
        lax.fori_loop(0, PB // 16, gd, 0)

    do_side(ui_hbm, rel1_u, rel2_u, e0_u, e1_u, e2_u, d_u)
    do_side(ii_hbm, rel1_i, rel2_i, e0_i, e1_i, e2_i, d_i)


def _sc_gather(user_index, item_index, adj_pad, ent, desc_tab):
    i32, f32 = jnp.int32, jnp.float32
    out_type = [
        jax.ShapeDtypeStruct((B * NNB,), i32),
        jax.ShapeDtypeStruct((B * NNB * NNB,), i32),
        jax.ShapeDtypeStruct((B * NNB,), i32),
        jax.ShapeDtypeStruct((B * NNB * NNB,), i32),
        jax.ShapeDtypeStruct((B, DIM), f32),
        jax.ShapeDtypeStruct((NNB, B, DIM), f32),
        jax.ShapeDtypeStruct((NNB * NNB, B, DIM), f32),
        jax.ShapeDtypeStruct((B, DIM), f32),
        jax.ShapeDtypeStruct((NNB, B, DIM), f32),
        jax.ShapeDtypeStruct((NNB * NNB, B, DIM), f32),
        jax.ShapeDtypeStruct((B, DESC_DIM), f32),
        jax.ShapeDtypeStruct((B, DESC_DIM), f32),
    ]
    scratch = [
        pltpu.VMEM((N_IDX,), i32),
        pltpu.VMEM((N_REL,), i32),
        pltpu.VMEM((PB, 128), i32),
        pltpu.VMEM((PB, DIM), f32),
        pltpu.VMEM((16, DESC_DIM), f32),
        pltpu.SemaphoreType.DMA,
        pltpu.SemaphoreType.DMA,
    ]
    mesh = plsc.VectorSubcoreMesh(core_axis_name="c", subcore_axis_name="s")
    fn = pl.kernel(_sc_body, out_type=out_type, mesh=mesh,
                   scratch_types=scratch)
    return fn(user_index, item_index, adj_pad, ent, desc_tab)


def _tc_body(e0u, e1u, e2u, e0i, e1i, e2i, r1u, r2u, r1i, r2i, du, di,
             rel, wagg, bagg, wdr, bdr, w1, b1, w2, b2, w3, b3, wnm, bnm,
             og, od):
    wt = wagg[0:DIM, :]
    wb = wagg[DIM:2 * DIM, :]
    ba = bagg[...]
    rel_m = rel[...]

    def gather_scores(s_rel, idx):
        out = jnp.zeros(idx.shape, jnp.float32)
        for k in range(32):
            out = out + jnp.where(idx == k, s_rel[:, k:k + 1], 0.0)
        return out

    def softmax_rows(s):
        m = jnp.max(s, axis=-1, keepdims=True)
        e = jnp.exp(s - m)
        return e / jnp.sum(e, axis=-1, keepdims=True)

    def aggregate(side, e0, e1_ref, e2_ref, r1, r2):
        s_rel = lax.dot_general(side, rel_m, (((1,), (1,)), ((), ())),
                                preferred_element_type=jnp.float32)
        s2 = gather_scores(s_rel, r2)
        e1s = [e1_ref[i, :, :] for i in range(NNB)]
        aggs = []
        for i in range(NNB):
            w2g = softmax_rows(s2[:, NNB * i:NNB * (i + 1)])
            a = w2g[:, 0:1] * e2_ref[NNB * i, :, :]
            for n in range(1, NNB):
                a = a + w2g[:, n:n + 1] * e2_ref[NNB * i + n, :, :]
            aggs.append(a)
        x1 = jnp.concatenate(e1s, axis=0)
        a1 = jnp.concatenate(aggs, axis=0)
        h1 = jax.nn.sigmoid(
            jnp.dot(x1, wt, preferred_element_type=jnp.float32)
            + jnp.dot(a1, wb, preferred_element_type=jnp.float32) + ba)
        w1a = softmax_rows(gather_scores(s_rel, r1))
        agg1 = w1a[:, 0:1] * e1s[0]
        aggf = w1a[:, 0:1] * h1[0:BLK, :]
        for i in range(1, NNB):
            agg1 = agg1 + w1a[:, i:i + 1] * e1s[i]
            aggf = aggf + w1a[:, i:i + 1] * h1[BLK * i:BLK * (i + 1), :]
        h0 = jax.nn.sigmoid(
            jnp.dot(e0, wt, preferred_element_type=jnp.float32)
            + jnp.dot(agg1, wb, preferred_element_type=jnp.float32) + ba)
        return jnp.tanh(
            jnp.dot(h0, wt, preferred_element_type=jnp.float32)
            + jnp.dot(aggf, wb, preferred_element_type=jnp.float32) + ba)

    side_u = e0u[...]
    item_graph = aggregate(side_u, e0i[...], e1i, e2i, r1i[...], r2i[...])
    user_graph = aggregate(item_graph, side_u, e1u, e2u, r1u[...], r2u[...])
    og[...] = jax.nn.sigmoid(jnp.sum(user_graph * item_graph, axis=1))

    ud = jax.nn.relu(jnp.dot(du[...], wdr[...],
                             preferred_element_type=jnp.float32) + bdr[...])
    idd = jax.nn.relu(jnp.dot(di[...], wdr[...],
                              preferred_element_type=jnp.float32) + bdr[...])
    nl = jax.nn.relu(
        jnp.dot(ud, w1[0:DIM, :], preferred_element_type=jnp.float32)
        + jnp.dot(idd, w1[DIM:2 * DIM, :], preferred_element_type=jnp.float32)
        + b1[...])
    nl = jax.nn.relu(jnp.dot(nl, w2[...],
                             preferred_element_type=jnp.float32) + b2[...])
    nl = jax.nn.relu(jnp.dot(nl, w3[...],
                             preferred_element_type=jnp.float32) + b3[...])
    lmul = ud * idd
    sd = (jnp.sum(lmul * wnm[:, 0:DIM], axis=1)
          + jnp.sum(nl * wnm[:, DIM:DIM + DIM // 2], axis=1) + bnm[0, 0])
    od[...] = jax.nn.sigmoid(sd)


def _tc_compute(e0u, e1u, e2u, e0i, e1i, e2i, r1u, r2u, r1i, r2i, du, di,
                rel, wagg, bagg, wdr, bdr, w1, b1, w2, b2, w3, b3, wnm, bnm):
    f32 = jnp.float32

    def blk(shape, imap):
        return pl.BlockSpec(shape, imap)

    row = lambda i: (i, 0)
    whole = lambda i: (0, 0)
    nmaj = lambda i: (0, i, 0)
    in_specs = [
        blk((BLK, DIM), row), blk((NNB, BLK, DIM), nmaj),
        blk((NNB * NNB, BLK, DIM), nmaj),
        blk((BLK, DIM), row), blk((NNB, BLK, DIM), nmaj),
        blk((NNB * NNB, BLK, DIM), nmaj),
        blk((BLK, NNB), row), blk((BLK, NNB * NNB), row),
        blk((BLK, NNB), row), blk((BLK, NNB * NNB), row),
        blk((BLK, DESC_DIM), row), blk((BLK, DESC_DIM), row),
        blk((32, DIM), whole), blk((2 * DIM, DIM), whole),
        blk((1, DIM), whole), blk((DESC_DIM, DIM), whole),
        blk((1, DIM), whole), blk((2 * DIM, 2 * DIM), whole),
        blk((1, 2 * DIM), whole), blk((2 * DIM, DIM), whole),
        blk((1, DIM), whole), blk((DIM, DIM // 2), whole),
        blk((1, DIM // 2), whole), blk((1, DIM + DIM // 2), whole),
        blk((1, 1), whole),
    ]
    out_specs = [pl.BlockSpec((BLK,), lambda i: (i,)),
                 pl.BlockSpec((BLK,), lambda i: (i,))]
    out_shape = [jax.ShapeDtypeStruct((B,), f32),
                 jax.ShapeDtypeStruct((B,), f32)]
    return pl.pallas_call(
        _tc_body, grid=(NB,), in_specs=in_specs, out_specs=out_specs,
        out_shape=out_shape,
    )(e0u, e1u, e2u, e0i, e1i, e2i, r1u, r2u, r1i, r2i, du, di,
      rel, wagg, bagg, wdr, bdr, w1, b1, w2, b2, w3, b3, wnm, bnm)


def kernel(user_index, item_index, adj_ent, adj_rel, ent, rel, desc_tab,
           W_agg, b_agg, W_dr, b_dr, W1, b1, W2, b2, W3, b3, W_nm, b_nm):
    num_ent = adj_ent.shape[0]
    adj_pad = jnp.concatenate(
        [adj_ent, adj_rel,
         jnp.zeros((num_ent, 128 - 2 * NNB), jnp.int32)], axis=1)
    (rel1_u, rel2_u, rel1_i, rel2_i,
     e0u, e1u, e2u, e0i, e1i, e2i, du, di) = _sc_gather(
        user_index, item_index, adj_pad, ent, desc_tab)
    r1u = rel1_u.reshape(B, NNB)
    r2u = rel2_u.reshape(B, NNB * NNB)
    r1i = rel1_i.reshape(B, NNB)
    r2i = rel2_i.reshape(B, NNB * NNB)
    og, od = _tc_compute(
        e0u, e1u, e2u, e0i, e1i, e2i, r1u, r2u, r1i, r2i, du, di,
        rel, W_agg, b_agg.reshape(1, DIM), W_dr, b_dr.reshape(1, DIM),
        W1, b1.reshape(1, 2 * DIM), W2, b2.reshape(1, DIM),
        W3, b3.reshape(1, DIM // 2), W_nm.reshape(1, DIM + DIM // 2),
        b_nm.reshape(1, 1))
    return og, od
